# baseline (speedup 1.0000x reference)
import functools

import jax
import jax.numpy as jnp
from jax import lax
from jax.experimental import pallas as pl
from jax.experimental.pallas import tpu as pltpu
from jax.experimental.pallas import tpu_sc as plsc

V = 100000
E = 256
H = 512
D = 512
B = 128
L = 128
B3 = 3 * B
N_TOK = L * B3


def _gather_sc(table, idx):
    info = plsc.get_sparse_core_info()
    nw = info.num_cores * info.num_subcores
    bpw = N_TOK // nw
    ch = 128
    nchunk = bpw // ch
    mesh = plsc.VectorSubcoreMesh(core_axis_name="c", subcore_axis_name="s")

    @functools.partial(
        pl.kernel,
        mesh=mesh,
        out_type=jax.ShapeDtypeStruct((N_TOK, E), jnp.float32),
        scratch_types=[
            pltpu.VMEM((bpw,), jnp.int32),
            pltpu.VMEM((ch, E), jnp.float32),
            pltpu.VMEM((ch, E), jnp.float32),
            pltpu.SemaphoreType.DMA,
            pltpu.SemaphoreType.DMA,
        ],
    )
    def k(table_hbm, idx_hbm, out_hbm, idx_v, rows0, rows1, sem0, sem1):
        cid = lax.axis_index("c")
        sid = lax.axis_index("s")
        wid = sid * info.num_cores + cid
        base = wid * bpw
        pltpu.sync_copy(idx_hbm.at[pl.ds(base, bpw)], idx_v)

        def body(i, carry):
            pltpu.async_copy(
                table_hbm.at[idx_v.at[pl.ds(i * ch, ch)]], rows0, sem0
            ).wait()
            pltpu.sync_copy(rows0, out_hbm.at[pl.ds(base + i * ch, ch)])
            return carry

        lax.fori_loop(0, nchunk, body, 0)

    return k(table, idx)


def _lstm_body(x_ref, W_ref, U_ref, b_ref, W1_ref, b1_ref, W2_ref, b2_ref,
               out_ref, h_scr, c_scr):
    t = pl.program_id(0)

    @pl.when(t == 0)
    def _init():
        h_scr[...] = jnp.zeros_like(h_scr)
        c_scr[...] = jnp.zeros_like(c_scr)

    xt = x_ref[0].astype(jnp.bfloat16)
    h = h_scr[...].astype(jnp.bfloat16)
    z = (jnp.dot(xt, W_ref[...], preferred_element_type=jnp.float32)
         + jnp.dot(h, U_ref[...], preferred_element_type=jnp.float32)
         + b_ref[...])
    i = jax.nn.sigmoid(z[:, :H])
    f = jax.nn.sigmoid(z[:, H:2 * H])
    g = jnp.tanh(z[:, 2 * H:3 * H])
    o = jax.nn.sigmoid(z[:, 3 * H:])
    c = f * c_scr[...] + i * g
    h_new = o * jnp.tanh(c)
    c_scr[...] = c
    h_scr[...] = h_new

    @pl.when(t == L - 1)
    def _head():
        h_a = h_new[:B]
        h_p = h_new[B:2 * B]
        h_n = h_new[2 * B:]
        hcat = jnp.concatenate(
            [jnp.concatenate([h_a, h_p], axis=1),
             jnp.concatenate([h_a, h_n], axis=1)], axis=0)
        h1 = jnp.maximum(
            jnp.dot(hcat, W1_ref[...], preferred_element_type=jnp.float32)
            + b1_ref[...], 0.0)
        s = jax.nn.sigmoid(
            jnp.dot(h1, W2_ref[...], preferred_element_type=jnp.float32)
            + b2_ref[0, 0])
        out_ref[...] = jnp.concatenate([s[:B], s[B:]], axis=1)


def _lstm_tc(x, W, U, b, W1, b1, W2, b2):
    full = lambda shape: pl.BlockSpec(shape, lambda t: (0,) * len(shape))
    return pl.pallas_call(
        _lstm_body,
        grid=(L,),
        in_specs=[
            pl.BlockSpec((1, B3, E), lambda t: (t, 0, 0)),
            full((E, 4 * H)),
            full((H, 4 * H)),
            full((1, 4 * H)),
            full((2 * H, D)),
            full((1, D)),
            full((D, 1)),
            pl.BlockSpec(memory_space=pltpu.SMEM),
        ],
        out_specs=pl.BlockSpec((B, 2), lambda t: (0, 0)),
        out_shape=jax.ShapeDtypeStruct((B, 2), jnp.float32),
        scratch_shapes=[
            pltpu.VMEM((B3, H), jnp.float32),
            pltpu.VMEM((B3, H), jnp.float32),
        ],
        compiler_params=pltpu.CompilerParams(
            dimension_semantics=("arbitrary",)),
    )(x, W, U, b, W1, b1, W2, b2)


def kernel(anchor_tokens, pos_tokens, neg_tokens, emb_table, W, U, b,
           W1, b1, W2, b2):
    toks = jnp.stack([anchor_tokens.astype(jnp.int32),
                      pos_tokens.astype(jnp.int32),
                      neg_tokens.astype(jnp.int32)])
    idx = toks.reshape(B3, L).T.reshape(-1)
    x = _gather_sc(emb_table, idx).reshape(L, B3, E)
    return _lstm_tc(x, W.astype(jnp.bfloat16), U.astype(jnp.bfloat16),
                    b.reshape(1, 4 * H), W1, b1.reshape(1, D),
                    W2, b2.reshape(1, 1))


# speedup vs baseline: 1.1575x; 1.1575x over previous
import functools

import jax
import jax.numpy as jnp
from jax import lax
from jax.experimental import pallas as pl
from jax.experimental.pallas import tpu as pltpu
from jax.experimental.pallas import tpu_sc as plsc

V = 100000
E = 256
H = 512
D = 512
B = 128
L = 128
B3 = 3 * B
N_TOK = L * B3


def _gather_sc(table, idx):
    info = plsc.get_sparse_core_info()
    nw = info.num_cores * info.num_subcores
    bpw = N_TOK // nw
    ch = 128
    nchunk = bpw // ch
    mesh = plsc.VectorSubcoreMesh(core_axis_name="c", subcore_axis_name="s")

    @functools.partial(
        pl.kernel,
        mesh=mesh,
        out_type=jax.ShapeDtypeStruct((N_TOK, E), jnp.float32),
        scratch_types=[
            pltpu.VMEM((bpw,), jnp.int32),
            pltpu.VMEM((ch, E), jnp.float32),
            pltpu.VMEM((ch, E), jnp.float32),
            pltpu.SemaphoreType.DMA,
            pltpu.SemaphoreType.DMA,
        ],
    )
    def k(table_hbm, idx_hbm, out_hbm, idx_v, rows0, rows1, sem0, sem1):
        cid = lax.axis_index("c")
        sid = lax.axis_index("s")
        wid = sid * info.num_cores + cid
        base = wid * bpw
        pltpu.sync_copy(idx_hbm.at[pl.ds(base, bpw)], idx_v)

        def body(i, carry):
            pltpu.async_copy(
                table_hbm.at[idx_v.at[pl.ds(i * ch, ch)]], rows0, sem0
            ).wait()
            pltpu.sync_copy(rows0, out_hbm.at[pl.ds(base + i * ch, ch)])
            return carry

        lax.fori_loop(0, nchunk, body, 0)

    return k(table, idx)


def _lstm_body(x_ref, W_ref, U_ref, b_ref, W1_ref, b1_ref, W2_ref, b2_ref,
               out_ref, h_scr, c_scr):
    t = pl.program_id(0)

    @pl.when(t == 0)
    def _init():
        h_scr[...] = jnp.zeros_like(h_scr)
        c_scr[...] = jnp.zeros_like(c_scr)

    xt = x_ref[0].astype(jnp.bfloat16)
    h = h_scr[...].astype(jnp.bfloat16)

    def zchunk(k):
        return (jnp.dot(xt, W_ref[:, k * H:(k + 1) * H],
                        preferred_element_type=jnp.float32)
                + jnp.dot(h, U_ref[:, k * H:(k + 1) * H],
                          preferred_element_type=jnp.float32)
                + b_ref[:, k * H:(k + 1) * H])

    def sig(v):
        return 0.5 * jnp.tanh(0.5 * v) + 0.5

    i = sig(zchunk(0))
    f = sig(zchunk(1))
    g = jnp.tanh(zchunk(2))
    o = sig(zchunk(3))
    c = f * c_scr[...] + i * g
    h_new = o * jnp.tanh(c)
    c_scr[...] = c
    h_scr[...] = h_new

    @pl.when(t == L - 1)
    def _head():
        h_a = h_new[:B]
        h_p = h_new[B:2 * B]
        h_n = h_new[2 * B:]
        hcat = jnp.concatenate(
            [jnp.concatenate([h_a, h_p], axis=1),
             jnp.concatenate([h_a, h_n], axis=1)], axis=0)
        h1 = jnp.maximum(
            jnp.dot(hcat, W1_ref[...], preferred_element_type=jnp.float32)
            + b1_ref[...], 0.0)
        s = jax.nn.sigmoid(
            jnp.dot(h1, W2_ref[...], preferred_element_type=jnp.float32)
            + b2_ref[0, 0])
        out_ref[...] = jnp.concatenate([s[:B], s[B:]], axis=1)


def _lstm_tc(x, W, U, b, W1, b1, W2, b2):
    full = lambda shape: pl.BlockSpec(shape, lambda t: (0,) * len(shape))
    return pl.pallas_call(
        _lstm_body,
        grid=(L,),
        in_specs=[
            pl.BlockSpec((1, B3, E), lambda t: (t, 0, 0)),
            full((E, 4 * H)),
            full((H, 4 * H)),
            full((1, 4 * H)),
            full((2 * H, D)),
            full((1, D)),
            full((D, 1)),
            pl.BlockSpec(memory_space=pltpu.SMEM),
        ],
        out_specs=pl.BlockSpec((B, 2), lambda t: (0, 0)),
        out_shape=jax.ShapeDtypeStruct((B, 2), jnp.float32),
        scratch_shapes=[
            pltpu.VMEM((B3, H), jnp.float32),
            pltpu.VMEM((B3, H), jnp.float32),
        ],
        compiler_params=pltpu.CompilerParams(
            dimension_semantics=("arbitrary",)),
    )(x, W, U, b, W1, b1, W2, b2)


def kernel(anchor_tokens, pos_tokens, neg_tokens, emb_table, W, U, b,
           W1, b1, W2, b2):
    toks = jnp.stack([anchor_tokens.astype(jnp.int32),
                      pos_tokens.astype(jnp.int32),
                      neg_tokens.astype(jnp.int32)])
    idx = toks.reshape(B3, L).T.reshape(-1)
    x = _gather_sc(emb_table, idx).reshape(L, B3, E)
    return _lstm_tc(x, W.astype(jnp.bfloat16), U.astype(jnp.bfloat16),
                    b.reshape(1, 4 * H), W1, b1.reshape(1, D),
                    W2, b2.reshape(1, 1))
